# baseline (speedup 1.0000x reference)
import functools

import jax
import jax.numpy as jnp
from jax import lax
from jax.experimental import pallas as pl
from jax.experimental.pallas import tpu as pltpu
from jax.experimental.pallas import tpu_sc as plsc

VOCAB = 1000000
EMB = 32
NUM_SAMPLED = 64
BATCH = 16384

NUM_CORES = 2
NUM_SUBCORES = 16
NW = NUM_CORES * NUM_SUBCORES
BPW = BATCH // NW

TC_BLK = 512
TC_GRID = BATCH // TC_BLK


def _sc_gather_body(emb_hbm, nce_hbm, bias_hbm, idx_hbm, lab_hbm, samp_hbm,
                    embed_o, truew_o, trueb_o, sampw_o, sampb_o,
                    idx_v, lab_v, rows_a, rows_b, brow, sidx, srow, sbrow,
                    sem_a, sem_b, sem_c, sem_s, sem_t):
    wid = lax.axis_index("s") * NUM_CORES + lax.axis_index("c")
    base = wid * BPW
    pltpu.sync_copy(idx_hbm.at[pl.ds(base, BPW)], idx_v)
    pltpu.sync_copy(lab_hbm.at[pl.ds(base, BPW)], lab_v)
    ca = pltpu.async_copy(emb_hbm.at[idx_v], rows_a, sem_a)
    cb = pltpu.async_copy(nce_hbm.at[lab_v], rows_b, sem_b)
    cc = pltpu.async_copy(bias_hbm.at[lab_v], brow, sem_c)
    ca.wait()
    pltpu.sync_copy(rows_a, embed_o.at[pl.ds(base, BPW)])
    cb.wait()
    pltpu.sync_copy(rows_b, truew_o.at[pl.ds(base, BPW)])
    cc.wait()
    pltpu.sync_copy(brow, trueb_o.at[pl.ds(base, BPW)])

    @pl.when(wid == 0)
    def _sampled():
        pltpu.sync_copy(samp_hbm, sidx)
        cs = pltpu.async_copy(nce_hbm.at[sidx], srow, sem_s)
        ct = pltpu.async_copy(bias_hbm.at[sidx], sbrow, sem_t)
        cs.wait()
        pltpu.sync_copy(srow, sampw_o)
        ct.wait()
        pltpu.sync_copy(sbrow, sampb_o)


@functools.cache
def _sc_gather():
  return pl.kernel(
    _sc_gather_body,
    out_type=(
        jax.ShapeDtypeStruct((BATCH, EMB), jnp.float32),
        jax.ShapeDtypeStruct((BATCH, EMB), jnp.float32),
        jax.ShapeDtypeStruct((BATCH, 1), jnp.float32),
        jax.ShapeDtypeStruct((NUM_SAMPLED, EMB), jnp.float32),
        jax.ShapeDtypeStruct((NUM_SAMPLED, 1), jnp.float32),
    ),
    mesh=plsc.VectorSubcoreMesh(core_axis_name="c", subcore_axis_name="s",
                                num_cores=NUM_CORES,
                                num_subcores=NUM_SUBCORES),
    scratch_types=[
        pltpu.VMEM((BPW,), jnp.int32),
        pltpu.VMEM((BPW,), jnp.int32),
        pltpu.VMEM((BPW, EMB), jnp.float32),
        pltpu.VMEM((BPW, EMB), jnp.float32),
        pltpu.VMEM((BPW, 1), jnp.float32),
        pltpu.VMEM((NUM_SAMPLED,), jnp.int32),
        pltpu.VMEM((NUM_SAMPLED, EMB), jnp.float32),
        pltpu.VMEM((NUM_SAMPLED, 1), jnp.float32),
        pltpu.SemaphoreType.DMA,
        pltpu.SemaphoreType.DMA,
        pltpu.SemaphoreType.DMA,
        pltpu.SemaphoreType.DMA,
        pltpu.SemaphoreType.DMA,
    ],
    compiler_params=pltpu.CompilerParams(use_tc_tiling_on_sc=False),
  )


def _xent_pos(x):
    return jnp.maximum(x, 0.0) - x + jnp.log1p(jnp.exp(-jnp.abs(x)))


def _xent_neg(x):
    return jnp.maximum(x, 0.0) + jnp.log1p(jnp.exp(-jnp.abs(x)))


def _log_q(ids_f32):
    return (jnp.log(ids_f32 + 2.0) - jnp.log(ids_f32 + 1.0)) / jnp.log(
        float(VOCAB) + 1.0)


def _tc_loss_body(embed_ref, truew_ref, trueb_ref, lab_ref, sampw_ref,
                  sampb_ref, samp_ref, out_ref):
    i = pl.program_id(0)

    @pl.when(i == 0)
    def _init():
        out_ref[...] = jnp.zeros_like(out_ref)

    e = embed_ref[...]
    tw = truew_ref[...]
    tb = trueb_ref[...]
    lab = lab_ref[...].astype(jnp.float32)

    true_logits = (jnp.sum(e * tw, axis=1, keepdims=True) + tb
                   - jnp.log(_log_q(lab) * float(NUM_SAMPLED)))

    samp = samp_ref[...].astype(jnp.float32)
    logq_s = jnp.log(_log_q(samp) * float(NUM_SAMPLED))
    sampled_logits = lax.dot_general(
        e, sampw_ref[...], (((1,), (1,)), ((), ())),
        preferred_element_type=jnp.float32,
        precision=lax.Precision.HIGHEST)
    sampled_logits = sampled_logits + sampb_ref[...] - logq_s

    partial = jnp.sum(_xent_pos(true_logits)) + jnp.sum(_xent_neg(sampled_logits))
    out_ref[...] += jnp.full((1, 1), 1.0 / float(BATCH),
                             dtype=jnp.float32) * partial


def _tc_loss(embed, true_w, true_b, labels, sampled_w, sampled_b, samp):
    return pl.pallas_call(
        _tc_loss_body,
        grid=(TC_GRID,),
        in_specs=[
            pl.BlockSpec((TC_BLK, EMB), lambda i: (i, 0)),
            pl.BlockSpec((TC_BLK, EMB), lambda i: (i, 0)),
            pl.BlockSpec((TC_BLK, 1), lambda i: (i, 0)),
            pl.BlockSpec((TC_BLK, 1), lambda i: (i, 0)),
            pl.BlockSpec((NUM_SAMPLED, EMB), lambda i: (0, 0)),
            pl.BlockSpec((1, NUM_SAMPLED), lambda i: (0, 0)),
            pl.BlockSpec((1, NUM_SAMPLED), lambda i: (0, 0)),
        ],
        out_specs=pl.BlockSpec((1, 1), lambda i: (0, 0)),
        out_shape=jax.ShapeDtypeStruct((1, 1), jnp.float32),
    )(embed, true_w, true_b, labels, sampled_w, sampled_b, samp)


def kernel(inputs, train_labels, embeddings, nce_weights, nce_biases):
    inputs = inputs.astype(jnp.int32)
    labels = train_labels.reshape(-1).astype(jnp.int32)

    u = jax.random.uniform(jax.random.key(42), (NUM_SAMPLED,))
    s = jnp.floor(jnp.exp(u * jnp.log(float(VOCAB) + 1.0))) - 1.0
    samp = jnp.clip(s, 0, VOCAB - 1).astype(jnp.int32)

    bias_2d = nce_biases.reshape(VOCAB, 1)

    embed, true_w, true_b, sampled_w, sampled_b = _sc_gather()(
        embeddings, nce_weights, bias_2d, inputs, labels, samp)

    nce_cost = _tc_loss(embed, true_w, true_b,
                        labels.reshape(BATCH, 1), sampled_w,
                        sampled_b.reshape(1, NUM_SAMPLED),
                        samp.reshape(1, NUM_SAMPLED))

    return embed, nce_cost[0, 0]


# speedup vs baseline: 1.9086x; 1.9086x over previous
import functools

import jax
import jax.numpy as jnp
from jax import lax
from jax.experimental import pallas as pl
from jax.experimental.pallas import tpu as pltpu
from jax.experimental.pallas import tpu_sc as plsc

VOCAB = 1000000
EMB = 32
NUM_SAMPLED = 64
BATCH = 16384

NUM_CORES = 2
NUM_SUBCORES = 16
NW = NUM_CORES * NUM_SUBCORES
BPW = BATCH // NW

TC_BLK = 512
TC_GRID = BATCH // TC_BLK


def _sc_gather_body(emb_hbm, nce_hbm, bias_hbm, idx_hbm, lab_hbm, samp_hbm,
                    embed_o, truew_o, trueb_o, sampw_o, sampb_o,
                    idx_v, lab_v, rows_a, rows_b, brow, sidx, srow, sbrow,
                    sem_a, sem_b, sem_c, sem_s, sem_t):
    wid = lax.axis_index("s") * NUM_CORES + lax.axis_index("c")
    base = wid * BPW
    pltpu.sync_copy(idx_hbm.at[pl.ds(base, BPW)], idx_v)
    pltpu.sync_copy(lab_hbm.at[pl.ds(base, BPW)], lab_v)
    ca = pltpu.async_copy(emb_hbm.at[idx_v], rows_a, sem_a)
    cb = pltpu.async_copy(nce_hbm.at[lab_v], rows_b, sem_b)
    cc = pltpu.async_copy(bias_hbm.at[lab_v], brow, sem_c)
    ca.wait()
    pltpu.sync_copy(rows_a, embed_o.at[pl.ds(base, BPW)])
    cb.wait()
    pltpu.sync_copy(rows_b, truew_o.at[pl.ds(base, BPW)])
    cc.wait()
    pltpu.sync_copy(brow, trueb_o.at[pl.ds(base, BPW)])

    @pl.when(wid == 0)
    def _sampled():
        pltpu.sync_copy(samp_hbm, sidx)
        cs = pltpu.async_copy(nce_hbm.at[sidx], srow, sem_s)
        ct = pltpu.async_copy(bias_hbm.at[sidx], sbrow, sem_t)
        cs.wait()
        pltpu.sync_copy(srow, sampw_o)
        ct.wait()
        pltpu.sync_copy(sbrow, sampb_o)


@functools.cache
def _sc_gather():
  return pl.kernel(
    _sc_gather_body,
    out_type=(
        jax.ShapeDtypeStruct((BATCH, EMB), jnp.float32),
        jax.ShapeDtypeStruct((BATCH, EMB), jnp.float32),
        jax.ShapeDtypeStruct((BATCH,), jnp.float32),
        jax.ShapeDtypeStruct((NUM_SAMPLED, EMB), jnp.float32),
        jax.ShapeDtypeStruct((NUM_SAMPLED,), jnp.float32),
    ),
    mesh=plsc.VectorSubcoreMesh(core_axis_name="c", subcore_axis_name="s",
                                num_cores=NUM_CORES,
                                num_subcores=NUM_SUBCORES),
    scratch_types=[
        pltpu.VMEM((BPW,), jnp.int32),
        pltpu.VMEM((BPW,), jnp.int32),
        pltpu.VMEM((BPW, EMB), jnp.float32),
        pltpu.VMEM((BPW, EMB), jnp.float32),
        pltpu.VMEM((BPW,), jnp.float32),
        pltpu.VMEM((NUM_SAMPLED,), jnp.int32),
        pltpu.VMEM((NUM_SAMPLED, EMB), jnp.float32),
        pltpu.VMEM((NUM_SAMPLED,), jnp.float32),
        pltpu.SemaphoreType.DMA,
        pltpu.SemaphoreType.DMA,
        pltpu.SemaphoreType.DMA,
        pltpu.SemaphoreType.DMA,
        pltpu.SemaphoreType.DMA,
    ],
    compiler_params=pltpu.CompilerParams(use_tc_tiling_on_sc=False),
  )


def _xent_pos(x):
    return jnp.maximum(x, 0.0) - x + jnp.log1p(jnp.exp(-jnp.abs(x)))


def _xent_neg(x):
    return jnp.maximum(x, 0.0) + jnp.log1p(jnp.exp(-jnp.abs(x)))


def _log_q(ids_f32):
    return (jnp.log(ids_f32 + 2.0) - jnp.log(ids_f32 + 1.0)) / jnp.log(
        float(VOCAB) + 1.0)


def _tc_loss_body(embed_ref, truew_ref, trueb_ref, lab_ref, sampw_ref,
                  sampb_ref, samp_ref, out_ref):
    i = pl.program_id(0)

    @pl.when(i == 0)
    def _init():
        out_ref[...] = jnp.zeros_like(out_ref)

    e = embed_ref[...]
    tw = truew_ref[...]
    tb = trueb_ref[...]
    lab = lab_ref[...].astype(jnp.float32)

    true_logits = (jnp.sum(e * tw, axis=1, keepdims=True) + tb
                   - jnp.log(_log_q(lab) * float(NUM_SAMPLED)))

    samp = samp_ref[...].astype(jnp.float32)
    logq_s = jnp.log(_log_q(samp) * float(NUM_SAMPLED))
    sampled_logits = lax.dot_general(
        e, sampw_ref[...], (((1,), (1,)), ((), ())),
        preferred_element_type=jnp.float32,
        precision=lax.Precision.HIGHEST)
    sampled_logits = sampled_logits + sampb_ref[...] - logq_s

    partial = jnp.sum(_xent_pos(true_logits)) + jnp.sum(_xent_neg(sampled_logits))
    out_ref[...] += jnp.full((1, 1), 1.0 / float(BATCH),
                             dtype=jnp.float32) * partial


def _tc_loss(embed, true_w, true_b, labels, sampled_w, sampled_b, samp):
    return pl.pallas_call(
        _tc_loss_body,
        grid=(TC_GRID,),
        in_specs=[
            pl.BlockSpec((TC_BLK, EMB), lambda i: (i, 0)),
            pl.BlockSpec((TC_BLK, EMB), lambda i: (i, 0)),
            pl.BlockSpec((TC_BLK, 1), lambda i: (i, 0)),
            pl.BlockSpec((TC_BLK, 1), lambda i: (i, 0)),
            pl.BlockSpec((NUM_SAMPLED, EMB), lambda i: (0, 0)),
            pl.BlockSpec((1, NUM_SAMPLED), lambda i: (0, 0)),
            pl.BlockSpec((1, NUM_SAMPLED), lambda i: (0, 0)),
        ],
        out_specs=pl.BlockSpec((1, 1), lambda i: (0, 0)),
        out_shape=jax.ShapeDtypeStruct((1, 1), jnp.float32),
    )(embed, true_w, true_b, labels, sampled_w, sampled_b, samp)


def kernel(inputs, train_labels, embeddings, nce_weights, nce_biases):
    inputs = inputs.astype(jnp.int32)
    labels = train_labels.reshape(-1).astype(jnp.int32)

    u = jax.random.uniform(jax.random.key(42), (NUM_SAMPLED,))
    s = jnp.floor(jnp.exp(u * jnp.log(float(VOCAB) + 1.0))) - 1.0
    samp = jnp.clip(s, 0, VOCAB - 1).astype(jnp.int32)

    embed, true_w, true_b, sampled_w, sampled_b = _sc_gather()(
        embeddings, nce_weights, nce_biases, inputs, labels, samp)

    nce_cost = _tc_loss(embed, true_w, true_b.reshape(BATCH, 1),
                        labels.reshape(BATCH, 1), sampled_w,
                        sampled_b.reshape(1, NUM_SAMPLED),
                        samp.reshape(1, NUM_SAMPLED))

    return embed, nce_cost[0, 0]
